# baseline (speedup 1.0000x reference)
import functools

import jax
import jax.numpy as jnp
from jax import lax
from jax.experimental import pallas as pl
from jax.experimental.pallas import tpu as pltpu
from jax.experimental.pallas import tpu_sc as plsc

N = 10000
E = 320000
D = 128

NC = 2
NS = 16
NW = NC * NS
EB = 128
NBLK = 160
EPS = NBLK * EB
E_PAD = NS * EPS
NBUF = 4
NQUAD = NBLK // NBUF
DEG_BLK = NBLK // NC
NPAD = 10240
ROWS_PER_TILE = NPAD // NS
ZROWS = 128
DEG_PER_TILE = NPAD // NS

RB = 1000
NB = N // RB

_MESH = dict(core_axis_name="c", subcore_axis_name="s",
             num_cores=NC, num_subcores=NS)
_SC_PARAMS = pltpu.CompilerParams(use_tc_tiling_on_sc=False)


def _deg_body(eidx_hbm, out_hbm, sidx_v, ones_v, stage_v, acc_sh):
    c = lax.axis_index("c")
    s = lax.axis_index("s")
    pltpu.sync_copy(eidx_hbm.at[0, s, pl.ds(c * DEG_BLK, DEG_BLK)], sidx_v)
    for j in range(128 // 16):
        ones_v[pl.ds(j * 16, 16)] = jnp.ones((16,), jnp.float32)
    for j in range(DEG_PER_TILE // 16):
        stage_v[pl.ds(j * 16, 16)] = jnp.zeros((16,), jnp.float32)
    pltpu.sync_copy(stage_v, acc_sh.at[pl.ds(s * DEG_PER_TILE, DEG_PER_TILE)])
    plsc.subcore_barrier()

    def step(j, carry):
        pltpu.sync_copy(ones_v, acc_sh.at[sidx_v.at[j]], add=True)
        return carry

    lax.fori_loop(0, DEG_BLK, step, 0)
    plsc.subcore_barrier()
    pltpu.sync_copy(acc_sh.at[pl.ds(s * DEG_PER_TILE, DEG_PER_TILE)], stage_v)
    pltpu.sync_copy(stage_v, out_hbm.at[c, pl.ds(s * DEG_PER_TILE, DEG_PER_TILE)])


_deg_call = pl.kernel(
    _deg_body,
    out_type=jax.ShapeDtypeStruct((NC, NPAD), jnp.float32),
    mesh=plsc.VectorSubcoreMesh(**_MESH),
    scratch_types=[
        pltpu.VMEM((DEG_BLK, EB), jnp.int32),
        pltpu.VMEM((EB,), jnp.float32),
        pltpu.VMEM((DEG_PER_TILE,), jnp.float32),
        pltpu.VMEM_SHARED((NPAD,), jnp.float32),
    ],
    compiler_params=_SC_PARAMS,
)


DH = D // 2


def _spmm_body(feat_hbm, eidx_hbm, out_hbm,
               sidx_v, didx_v, rows_v, zbuf_v, acc_sh, *sems):
    gsem = sems[:NBUF]
    ssem = sems[NBUF:]
    c = lax.axis_index("c")
    s = lax.axis_index("s")
    pltpu.sync_copy(eidx_hbm.at[0, s], sidx_v)
    pltpu.sync_copy(eidx_hbm.at[1, s], didx_v)
    feat_c = feat_hbm.at[c]

    def gissue(j, b):
        pltpu.async_copy(feat_c.at[sidx_v.at[j]], rows_v.at[b], gsem[b])

    def zrow(i, carry):
        for j in range(DH // 16):
            zbuf_v[i, pl.ds(j * 16, 16)] = jnp.zeros((16,), jnp.float32)
        return carry

    lax.fori_loop(0, ZROWS, zrow, 0)
    for k in range(ROWS_PER_TILE // ZROWS):
        pltpu.sync_copy(
            zbuf_v, acc_sh.at[pl.ds(s * ROWS_PER_TILE + k * ZROWS, ZROWS)])
    plsc.subcore_barrier()

    for b in range(NBUF):
        gissue(b, b)

    def quad(j4, carry):
        j = j4 * NBUF
        sdesc = []
        for b in range(NBUF):
            pltpu.make_async_copy(
                feat_c.at[sidx_v.at[j + b]], rows_v.at[b],
                gsem[b]).wait()
            sdesc.append(pltpu.async_copy(
                rows_v.at[b], acc_sh.at[didx_v.at[j + b]], ssem[b],
                add=True))
        for b in range(NBUF):
            sdesc[b].wait()

            @pl.when(j4 < NQUAD - 1)
            def _(b=b):
                gissue(j + NBUF + b, b)
        return carry

    lax.fori_loop(0, NQUAD, quad, 0)
    plsc.subcore_barrier()
    for k in range(ROWS_PER_TILE // ZROWS):
        r0 = s * ROWS_PER_TILE + k * ZROWS
        pltpu.sync_copy(acc_sh.at[pl.ds(r0, ZROWS)], zbuf_v)
        pltpu.sync_copy(zbuf_v, out_hbm.at[c, pl.ds(r0, ZROWS)])


_spmm_call = pl.kernel(
    _spmm_body,
    out_type=jax.ShapeDtypeStruct((NC, NPAD, DH), jnp.float32),
    mesh=plsc.VectorSubcoreMesh(**_MESH),
    scratch_types=[
        pltpu.VMEM((NBLK, EB), jnp.int32),
        pltpu.VMEM((NBLK, EB), jnp.int32),
        pltpu.VMEM((NBUF, EB, DH), jnp.float32),
        pltpu.VMEM((ZROWS, DH), jnp.float32),
        pltpu.VMEM_SHARED((NPAD, DH), jnp.float32),
    ] + [pltpu.SemaphoreType.DMA] * (2 * NBUF),
    compiler_params=_SC_PARAMS,
)


def _mid1_body(u_ref, dis_ref, zin_ref, w_ref, z_ref, stats_ref, ssum, ssq):
    i = pl.program_id(0)
    t = -dis_ref[...] * jnp.concatenate([u_ref[0], u_ref[1]], axis=1)
    z = zin_ref[...] + _dot(t, w_ref[...])
    z_ref[...] = z

    @pl.when(i == 0)
    def _():
        ssum[...] = jnp.zeros_like(ssum)
        ssq[...] = jnp.zeros_like(ssq)

    ssum[...] += jnp.sum(z, axis=0, keepdims=True)
    ssq[...] += jnp.sum(z * z, axis=0, keepdims=True)

    @pl.when(i == NB - 1)
    def _():
        stats_ref[...] = jnp.concatenate([ssum[...], ssq[...]], axis=0)


def _mid1_call(u, dis, zin, W1_1):
    return pl.pallas_call(
        _mid1_body,
        grid=(NB,),
        in_specs=[
            pl.BlockSpec((NC, RB, DH), lambda i: (0, i, 0)),
            pl.BlockSpec((RB, 1), lambda i: (i, 0)),
            pl.BlockSpec((RB, D), lambda i: (i, 0)),
            pl.BlockSpec((D, D), lambda i: (0, 0)),
        ],
        out_specs=[
            pl.BlockSpec((RB, D), lambda i: (i, 0)),
            pl.BlockSpec((2, D), lambda i: (0, 0)),
        ],
        out_shape=[
            jax.ShapeDtypeStruct((N, D), jnp.float32),
            jax.ShapeDtypeStruct((2, D), jnp.float32),
        ],
        scratch_shapes=[
            pltpu.VMEM((1, D), jnp.float32),
            pltpu.VMEM((1, D), jnp.float32),
        ],
        compiler_params=pltpu.CompilerParams(
            dimension_semantics=("arbitrary",)),
    )(u, dis, zin, W1_1)


def _fin_body(v_ref, dis_ref, hw_ref, w_ref, mu_ref, ls_ref):
    t = -dis_ref[...] * jnp.concatenate([v_ref[0], v_ref[1]], axis=1)
    cat = hw_ref[...] + _dot(t, w_ref[...])
    mu_ref[...] = cat[:, :DH]
    ls_ref[...] = cat[:, DH:]


def _fin_call(v, dis, hW, Wcat2):
    return pl.pallas_call(
        _fin_body,
        grid=(NB,),
        in_specs=[
            pl.BlockSpec((NC, RB, DH), lambda i: (0, i, 0)),
            pl.BlockSpec((RB, 1), lambda i: (i, 0)),
            pl.BlockSpec((RB, D), lambda i: (i, 0)),
            pl.BlockSpec((D, D), lambda i: (0, 0)),
        ],
        out_specs=[
            pl.BlockSpec((RB, DH), lambda i: (i, 0)),
            pl.BlockSpec((RB, DH), lambda i: (i, 0)),
        ],
        out_shape=[
            jax.ShapeDtypeStruct((N, DH), jnp.float32),
            jax.ShapeDtypeStruct((N, DH), jnp.float32),
        ],
    )(v, dis, hW, Wcat2)


def _dot(a, b):
    return jnp.dot(a, b, preferred_element_type=jnp.float32)


def _prep_body(deg_ref, x_ref, dis_ref, xp_ref):
    degb = deg_ref[0, 0] + deg_ref[1, 0]
    pos = degb > 0.0
    dis = jnp.where(pos, lax.rsqrt(jnp.where(pos, degb, 1.0)), 0.0)
    dis_ref[...] = dis
    xp = dis * x_ref[...]
    xp_ref[0] = xp[:, :DH]
    xp_ref[1] = xp[:, DH:]


def _prep_call(degp, x):
    deg4 = degp[:, :N].reshape(NC, NB, RB, 1)
    return pl.pallas_call(
        _prep_body,
        grid=(NB,),
        in_specs=[
            pl.BlockSpec((NC, 1, RB, 1), lambda i: (0, i, 0, 0)),
            pl.BlockSpec((RB, D), lambda i: (i, 0)),
        ],
        out_specs=[
            pl.BlockSpec((RB, 1), lambda i: (i, 0)),
            pl.BlockSpec((NC, RB, DH), lambda i: (0, i, 0)),
        ],
        out_shape=[
            jax.ShapeDtypeStruct((N, 1), jnp.float32),
            jax.ShapeDtypeStruct((NC, NPAD, DH), jnp.float32),
        ],
    )(deg4, x)


def _xw0_body(x_ref, w_ref, b_ref, z_ref):
    z_ref[...] = _dot(x_ref[...], w_ref[...]) + b_ref[...]


def _xw0_call(x, W1_0, b1r):
    return pl.pallas_call(
        _xw0_body,
        grid=(NB,),
        in_specs=[
            pl.BlockSpec((RB, D), lambda i: (i, 0)),
            pl.BlockSpec((D, D), lambda i: (0, 0)),
            pl.BlockSpec((1, D), lambda i: (0, 0)),
        ],
        out_specs=pl.BlockSpec((RB, D), lambda i: (i, 0)),
        out_shape=jax.ShapeDtypeStruct((N, D), jnp.float32),
    )(x, W1_0, b1r)


def _bn_h(z_ref, stats_ref, g_ref, bt_ref):
    inv_n = jnp.float32(1.0 / N)
    mean = stats_ref[0:1, :] * inv_n
    var = stats_ref[1:2, :] * inv_n - mean * mean
    inv = lax.rsqrt(var + 1e-5)
    h = (z_ref[...] - mean) * inv * g_ref[...] + bt_ref[...]
    return jnp.maximum(h, 0.0)


def _bn_body(z_ref, stats_ref, dis_ref, g_ref, bt_ref, hp_ref):
    hp = dis_ref[...] * _bn_h(z_ref, stats_ref, g_ref, bt_ref)
    hp_ref[0] = hp[:, :DH]
    hp_ref[1] = hp[:, DH:]


def _bn_call(z, stats, dis, gamma2, beta2):
    return pl.pallas_call(
        _bn_body,
        grid=(NB,),
        in_specs=[
            pl.BlockSpec((RB, D), lambda i: (i, 0)),
            pl.BlockSpec((2, D), lambda i: (0, 0)),
            pl.BlockSpec((RB, 1), lambda i: (i, 0)),
            pl.BlockSpec((1, D), lambda i: (0, 0)),
            pl.BlockSpec((1, D), lambda i: (0, 0)),
        ],
        out_specs=pl.BlockSpec((NC, RB, DH), lambda i: (0, i, 0)),
        out_shape=jax.ShapeDtypeStruct((NC, NPAD, DH), jnp.float32),
    )(z, stats, dis, gamma2, beta2)


def _hw_body(z_ref, stats_ref, g_ref, bt_ref, wcat_ref, bcat_ref, hw_ref):
    h = _bn_h(z_ref, stats_ref, g_ref, bt_ref)
    hw_ref[...] = _dot(h, wcat_ref[...]) + bcat_ref[...]


def _hw_call(z, stats, gamma2, beta2, Wcat, bcat2):
    return pl.pallas_call(
        _hw_body,
        grid=(NB,),
        in_specs=[
            pl.BlockSpec((RB, D), lambda i: (i, 0)),
            pl.BlockSpec((2, D), lambda i: (0, 0)),
            pl.BlockSpec((1, D), lambda i: (0, 0)),
            pl.BlockSpec((1, D), lambda i: (0, 0)),
            pl.BlockSpec((D, D), lambda i: (0, 0)),
            pl.BlockSpec((1, D), lambda i: (0, 0)),
        ],
        out_specs=pl.BlockSpec((RB, D), lambda i: (i, 0)),
        out_shape=jax.ShapeDtypeStruct((N, D), jnp.float32),
    )(z, stats, gamma2, beta2, Wcat, bcat2)


def kernel(x, edge_index, W1_0, W1_1, b1, gamma, beta,
           Wmu_0, Wmu_1, b_mu, Wls_0, Wls_1, b_ls):
    ei3 = edge_index.astype(jnp.int32).reshape(2, NS, E // NS)
    npad_slab = EPS - E // NS
    tvals = (N + (jnp.arange(npad_slab) % (NPAD - N))).astype(jnp.int32)
    trash = jnp.broadcast_to(tvals, (2, NS, npad_slab))
    eidx = jnp.concatenate([ei3, trash], axis=2).reshape(2, NS, NBLK, EB)
    gamma2 = gamma.reshape(1, D)
    beta2 = beta.reshape(1, D)
    Wcat = jnp.concatenate([Wmu_0, Wls_0], axis=1)
    Wcat2 = jnp.concatenate([Wmu_1, Wls_1], axis=1)
    bcat2 = jnp.concatenate([b_mu, b_ls]).reshape(1, D)

    degp = _deg_call(eidx)
    dis, xp = _prep_call(degp, x)
    xw0 = _xw0_call(x, W1_0, b1.reshape(1, D))
    u = _spmm_call(xp, eidx)
    z, stats = _mid1_call(u, dis, xw0, W1_1)
    hp = _bn_call(z, stats, dis, gamma2, beta2)
    v = _spmm_call(hp, eidx)
    hW = _hw_call(z, stats, gamma2, beta2, Wcat, bcat2)
    return _fin_call(v, dis, hW, Wcat2)


# speedup vs baseline: 1.0121x; 1.0121x over previous
import functools

import jax
import jax.numpy as jnp
from jax import lax
from jax.experimental import pallas as pl
from jax.experimental.pallas import tpu as pltpu
from jax.experimental.pallas import tpu_sc as plsc

N = 10000
E = 320000
D = 128

NC = 2
NS = 16
NW = NC * NS
EB = 128
NBLK = 160
EPS = NBLK * EB
E_PAD = NS * EPS
NBUF = 5
NQUAD = NBLK // NBUF
DEG_BLK = NBLK // NC
NPAD = 10240
ROWS_PER_TILE = NPAD // NS
ZROWS = 64
DEG_PER_TILE = NPAD // NS

RB = 1000
NB = N // RB

_MESH = dict(core_axis_name="c", subcore_axis_name="s",
             num_cores=NC, num_subcores=NS)
_SC_PARAMS = pltpu.CompilerParams(use_tc_tiling_on_sc=False)


def _deg_body(eidx_hbm, out_hbm, sidx_v, ones_v, stage_v, acc_sh):
    c = lax.axis_index("c")
    s = lax.axis_index("s")
    pltpu.sync_copy(eidx_hbm.at[0, s, pl.ds(c * DEG_BLK, DEG_BLK)], sidx_v)
    for j in range(128 // 16):
        ones_v[pl.ds(j * 16, 16)] = jnp.ones((16,), jnp.float32)
    for j in range(DEG_PER_TILE // 16):
        stage_v[pl.ds(j * 16, 16)] = jnp.zeros((16,), jnp.float32)
    pltpu.sync_copy(stage_v, acc_sh.at[pl.ds(s * DEG_PER_TILE, DEG_PER_TILE)])
    plsc.subcore_barrier()

    def step(j, carry):
        pltpu.sync_copy(ones_v, acc_sh.at[sidx_v.at[j]], add=True)
        return carry

    lax.fori_loop(0, DEG_BLK, step, 0)
    plsc.subcore_barrier()
    pltpu.sync_copy(acc_sh.at[pl.ds(s * DEG_PER_TILE, DEG_PER_TILE)], stage_v)
    pltpu.sync_copy(stage_v, out_hbm.at[c, pl.ds(s * DEG_PER_TILE, DEG_PER_TILE)])


_deg_call = pl.kernel(
    _deg_body,
    out_type=jax.ShapeDtypeStruct((NC, NPAD), jnp.float32),
    mesh=plsc.VectorSubcoreMesh(**_MESH),
    scratch_types=[
        pltpu.VMEM((DEG_BLK, EB), jnp.int32),
        pltpu.VMEM((EB,), jnp.float32),
        pltpu.VMEM((DEG_PER_TILE,), jnp.float32),
        pltpu.VMEM_SHARED((NPAD,), jnp.float32),
    ],
    compiler_params=_SC_PARAMS,
)


DH = D // 2


def _spmm_body(feat_hbm, eidx_hbm, out_hbm,
               sidx_v, didx_v, rows_v, zbuf_v, acc_sh, *sems):
    gsem = sems[:NBUF]
    ssem = sems[NBUF:]
    c = lax.axis_index("c")
    s = lax.axis_index("s")
    pltpu.sync_copy(eidx_hbm.at[0, s], sidx_v)
    pltpu.sync_copy(eidx_hbm.at[1, s], didx_v)
    feat_c = feat_hbm.at[c]

    def gissue(j, b):
        pltpu.async_copy(feat_c.at[sidx_v.at[j]], rows_v.at[b], gsem[b])

    def zrow(i, carry):
        for j in range(DH // 16):
            zbuf_v[i, pl.ds(j * 16, 16)] = jnp.zeros((16,), jnp.float32)
        return carry

    lax.fori_loop(0, ZROWS, zrow, 0)
    for k in range(ROWS_PER_TILE // ZROWS):
        pltpu.sync_copy(
            zbuf_v, acc_sh.at[pl.ds(s * ROWS_PER_TILE + k * ZROWS, ZROWS)])
    plsc.subcore_barrier()

    for b in range(NBUF):
        gissue(b, b)

    def quad(j4, carry):
        j = j4 * NBUF
        sdesc = []
        for b in range(NBUF):
            pltpu.make_async_copy(
                feat_c.at[sidx_v.at[j + b]], rows_v.at[b],
                gsem[b]).wait()
            sdesc.append(pltpu.async_copy(
                rows_v.at[b], acc_sh.at[didx_v.at[j + b]], ssem[b],
                add=True))
        for b in range(NBUF):
            sdesc[b].wait()

            @pl.when(j4 < NQUAD - 1)
            def _(b=b):
                gissue(j + NBUF + b, b)
        return carry

    lax.fori_loop(0, NQUAD, quad, 0)
    plsc.subcore_barrier()
    for k in range(ROWS_PER_TILE // ZROWS):
        r0 = s * ROWS_PER_TILE + k * ZROWS
        pltpu.sync_copy(acc_sh.at[pl.ds(r0, ZROWS)], zbuf_v)
        pltpu.sync_copy(zbuf_v, out_hbm.at[c, pl.ds(r0, ZROWS)])


_spmm_call = pl.kernel(
    _spmm_body,
    out_type=jax.ShapeDtypeStruct((NC, NPAD, DH), jnp.float32),
    mesh=plsc.VectorSubcoreMesh(**_MESH),
    scratch_types=[
        pltpu.VMEM((NBLK, EB), jnp.int32),
        pltpu.VMEM((NBLK, EB), jnp.int32),
        pltpu.VMEM((NBUF, EB, DH), jnp.float32),
        pltpu.VMEM((ZROWS, DH), jnp.float32),
        pltpu.VMEM_SHARED((NPAD, DH), jnp.float32),
    ] + [pltpu.SemaphoreType.DMA] * (2 * NBUF),
    compiler_params=_SC_PARAMS,
)


def _mid1_body(u_ref, dis_ref, zin_ref, w_ref, z_ref, stats_ref, ssum, ssq):
    i = pl.program_id(0)
    t = -dis_ref[...] * jnp.concatenate([u_ref[0], u_ref[1]], axis=1)
    z = zin_ref[...] + _dot(t, w_ref[...])
    z_ref[...] = z

    @pl.when(i == 0)
    def _():
        ssum[...] = jnp.zeros_like(ssum)
        ssq[...] = jnp.zeros_like(ssq)

    ssum[...] += jnp.sum(z, axis=0, keepdims=True)
    ssq[...] += jnp.sum(z * z, axis=0, keepdims=True)

    @pl.when(i == NB - 1)
    def _():
        stats_ref[...] = jnp.concatenate([ssum[...], ssq[...]], axis=0)


def _mid1_call(u, dis, zin, W1_1):
    return pl.pallas_call(
        _mid1_body,
        grid=(NB,),
        in_specs=[
            pl.BlockSpec((NC, RB, DH), lambda i: (0, i, 0)),
            pl.BlockSpec((RB, 1), lambda i: (i, 0)),
            pl.BlockSpec((RB, D), lambda i: (i, 0)),
            pl.BlockSpec((D, D), lambda i: (0, 0)),
        ],
        out_specs=[
            pl.BlockSpec((RB, D), lambda i: (i, 0)),
            pl.BlockSpec((2, D), lambda i: (0, 0)),
        ],
        out_shape=[
            jax.ShapeDtypeStruct((N, D), jnp.float32),
            jax.ShapeDtypeStruct((2, D), jnp.float32),
        ],
        scratch_shapes=[
            pltpu.VMEM((1, D), jnp.float32),
            pltpu.VMEM((1, D), jnp.float32),
        ],
        compiler_params=pltpu.CompilerParams(
            dimension_semantics=("arbitrary",)),
    )(u, dis, zin, W1_1)


def _fin_body(v_ref, dis_ref, hw_ref, w_ref, mu_ref, ls_ref):
    t = -dis_ref[...] * jnp.concatenate([v_ref[0], v_ref[1]], axis=1)
    cat = hw_ref[...] + _dot(t, w_ref[...])
    mu_ref[...] = cat[:, :DH]
    ls_ref[...] = cat[:, DH:]


def _fin_call(v, dis, hW, Wcat2):
    return pl.pallas_call(
        _fin_body,
        grid=(NB,),
        in_specs=[
            pl.BlockSpec((NC, RB, DH), lambda i: (0, i, 0)),
            pl.BlockSpec((RB, 1), lambda i: (i, 0)),
            pl.BlockSpec((RB, D), lambda i: (i, 0)),
            pl.BlockSpec((D, D), lambda i: (0, 0)),
        ],
        out_specs=[
            pl.BlockSpec((RB, DH), lambda i: (i, 0)),
            pl.BlockSpec((RB, DH), lambda i: (i, 0)),
        ],
        out_shape=[
            jax.ShapeDtypeStruct((N, DH), jnp.float32),
            jax.ShapeDtypeStruct((N, DH), jnp.float32),
        ],
    )(v, dis, hW, Wcat2)


def _dot(a, b):
    return jnp.dot(a, b, preferred_element_type=jnp.float32)


def _prep_body(deg_ref, x_ref, dis_ref, xp_ref):
    degb = deg_ref[0, 0] + deg_ref[1, 0]
    pos = degb > 0.0
    dis = jnp.where(pos, lax.rsqrt(jnp.where(pos, degb, 1.0)), 0.0)
    dis_ref[...] = dis
    xp = dis * x_ref[...]
    xp_ref[0] = xp[:, :DH]
    xp_ref[1] = xp[:, DH:]


def _prep_call(degp, x):
    deg4 = degp[:, :N].reshape(NC, NB, RB, 1)
    return pl.pallas_call(
        _prep_body,
        grid=(NB,),
        in_specs=[
            pl.BlockSpec((NC, 1, RB, 1), lambda i: (0, i, 0, 0)),
            pl.BlockSpec((RB, D), lambda i: (i, 0)),
        ],
        out_specs=[
            pl.BlockSpec((RB, 1), lambda i: (i, 0)),
            pl.BlockSpec((NC, RB, DH), lambda i: (0, i, 0)),
        ],
        out_shape=[
            jax.ShapeDtypeStruct((N, 1), jnp.float32),
            jax.ShapeDtypeStruct((NC, NPAD, DH), jnp.float32),
        ],
    )(deg4, x)


def _xw0_body(x_ref, w_ref, b_ref, z_ref):
    z_ref[...] = _dot(x_ref[...], w_ref[...]) + b_ref[...]


def _xw0_call(x, W1_0, b1r):
    return pl.pallas_call(
        _xw0_body,
        grid=(NB,),
        in_specs=[
            pl.BlockSpec((RB, D), lambda i: (i, 0)),
            pl.BlockSpec((D, D), lambda i: (0, 0)),
            pl.BlockSpec((1, D), lambda i: (0, 0)),
        ],
        out_specs=pl.BlockSpec((RB, D), lambda i: (i, 0)),
        out_shape=jax.ShapeDtypeStruct((N, D), jnp.float32),
    )(x, W1_0, b1r)


def _bn_h(z_ref, stats_ref, g_ref, bt_ref):
    inv_n = jnp.float32(1.0 / N)
    mean = stats_ref[0:1, :] * inv_n
    var = stats_ref[1:2, :] * inv_n - mean * mean
    inv = lax.rsqrt(var + 1e-5)
    h = (z_ref[...] - mean) * inv * g_ref[...] + bt_ref[...]
    return jnp.maximum(h, 0.0)


def _bn_body(z_ref, stats_ref, dis_ref, g_ref, bt_ref, hp_ref):
    hp = dis_ref[...] * _bn_h(z_ref, stats_ref, g_ref, bt_ref)
    hp_ref[0] = hp[:, :DH]
    hp_ref[1] = hp[:, DH:]


def _bn_call(z, stats, dis, gamma2, beta2):
    return pl.pallas_call(
        _bn_body,
        grid=(NB,),
        in_specs=[
            pl.BlockSpec((RB, D), lambda i: (i, 0)),
            pl.BlockSpec((2, D), lambda i: (0, 0)),
            pl.BlockSpec((RB, 1), lambda i: (i, 0)),
            pl.BlockSpec((1, D), lambda i: (0, 0)),
            pl.BlockSpec((1, D), lambda i: (0, 0)),
        ],
        out_specs=pl.BlockSpec((NC, RB, DH), lambda i: (0, i, 0)),
        out_shape=jax.ShapeDtypeStruct((NC, NPAD, DH), jnp.float32),
    )(z, stats, dis, gamma2, beta2)


def _hw_body(z_ref, stats_ref, g_ref, bt_ref, wcat_ref, bcat_ref, hw_ref):
    h = _bn_h(z_ref, stats_ref, g_ref, bt_ref)
    hw_ref[...] = _dot(h, wcat_ref[...]) + bcat_ref[...]


def _hw_call(z, stats, gamma2, beta2, Wcat, bcat2):
    return pl.pallas_call(
        _hw_body,
        grid=(NB,),
        in_specs=[
            pl.BlockSpec((RB, D), lambda i: (i, 0)),
            pl.BlockSpec((2, D), lambda i: (0, 0)),
            pl.BlockSpec((1, D), lambda i: (0, 0)),
            pl.BlockSpec((1, D), lambda i: (0, 0)),
            pl.BlockSpec((D, D), lambda i: (0, 0)),
            pl.BlockSpec((1, D), lambda i: (0, 0)),
        ],
        out_specs=pl.BlockSpec((RB, D), lambda i: (i, 0)),
        out_shape=jax.ShapeDtypeStruct((N, D), jnp.float32),
    )(z, stats, gamma2, beta2, Wcat, bcat2)


def kernel(x, edge_index, W1_0, W1_1, b1, gamma, beta,
           Wmu_0, Wmu_1, b_mu, Wls_0, Wls_1, b_ls):
    ei3 = edge_index.astype(jnp.int32).reshape(2, NS, E // NS)
    npad_slab = EPS - E // NS
    tvals = (N + (jnp.arange(npad_slab) % (NPAD - N))).astype(jnp.int32)
    trash = jnp.broadcast_to(tvals, (2, NS, npad_slab))
    eidx = jnp.concatenate([ei3, trash], axis=2).reshape(2, NS, NBLK, EB)
    gamma2 = gamma.reshape(1, D)
    beta2 = beta.reshape(1, D)
    Wcat = jnp.concatenate([Wmu_0, Wls_0], axis=1)
    Wcat2 = jnp.concatenate([Wmu_1, Wls_1], axis=1)
    bcat2 = jnp.concatenate([b_mu, b_ls]).reshape(1, D)

    degp = _deg_call(eidx)
    dis, xp = _prep_call(degp, x)
    xw0 = _xw0_call(x, W1_0, b1.reshape(1, D))
    u = _spmm_call(xp, eidx)
    z, stats = _mid1_call(u, dis, xw0, W1_1)
    hp = _bn_call(z, stats, dis, gamma2, beta2)
    v = _spmm_call(hp, eidx)
    hW = _hw_call(z, stats, gamma2, beta2, Wcat, bcat2)
    return _fin_call(v, dis, hW, Wcat2)
